# baseline (speedup 1.0000x reference)
import functools
import math

import numpy as np
import jax
import jax.numpy as jnp
from jax import lax
from jax.experimental import pallas as pl
from jax.experimental.pallas import tpu as pltpu
from jax.experimental.pallas import tpu_sc as plsc

VOCAB = 1000000
D = 64
P = 128
B = 1024
S = 200
SPAD = 256
NTOK = B * S
HALF = NTOK // 2

NC, NS = 2, 16
NW = NC * NS
ROWS_PER_W = B // NW
ROWS_PER_CHUNK = 4
NCHUNK = ROWS_PER_W // ROWS_PER_CHUNK
GCHUNK = ROWS_PER_CHUNK * SPAD
W_PER_HALF = NW // 2


def _sc_gather(table, idx_pad_flat):
    mesh = plsc.VectorSubcoreMesh(core_axis_name="c", subcore_axis_name="s")

    @functools.partial(
        pl.kernel,
        mesh=mesh,
        out_type=jax.ShapeDtypeStruct((HALF, P), jnp.float32),
        scratch_types=[
            pltpu.VMEM((GCHUNK,), jnp.int32),
            pltpu.VMEM((GCHUNK, D), jnp.float32),
            pltpu.SemaphoreType.DMA,
        ],
        compiler_params=pltpu.CompilerParams(use_tc_tiling_on_sc=False),
    )
    def k(table_hbm, idx_hbm, out_hbm, idx_v, rows_v, sem):
        wid = lax.axis_index("s") * NC + lax.axis_index("c")
        xrow0 = wid * ROWS_PER_W
        row_base = (wid % W_PER_HALF) * ROWS_PER_W * S

        def body(i, carry):
            xrow = xrow0 + i * ROWS_PER_CHUNK
            pltpu.sync_copy(idx_hbm.at[pl.ds(xrow * SPAD, GCHUNK)], idx_v)
            pltpu.async_copy(table_hbm.at[idx_v], rows_v, sem).wait()
            row0 = row_base + i * ROWS_PER_CHUNK * S

            def copy_rows(col):
                for g in range(ROWS_PER_CHUNK):
                    pltpu.sync_copy(
                        rows_v.at[pl.ds(g * SPAD, S)],
                        out_hbm.at[pl.ds(row0 + g * S, S), pl.ds(col, D)])

            @pl.when(wid < W_PER_HALF)
            def _():
                copy_rows(0)

            @pl.when(wid >= W_PER_HALF)
            def _():
                copy_rows(D)

            return carry

        lax.fori_loop(0, NCHUNK, body, 0)

    return k(table, idx_pad_flat)


TB = 1600
NSTEPS = HALF // TB


def _positional_encoding_np(seq_len, embed_dim):
    position = np.arange(0, seq_len, dtype=np.float32)[:, None]
    div_term = np.exp(
        np.arange(0, embed_dim, 2, dtype=np.float32)
        * (-math.log(10000.0) / embed_dim))
    pe = np.zeros((seq_len, embed_dim), dtype=np.float32)
    pe[:, 0::2] = np.sin(position * div_term)
    pe[:, 1::2] = np.cos(position * div_term)
    return pe


_PE_TILE = np.tile(_positional_encoding_np(S, D), (TB // S, 1))


def _tc_ln_proj(emb2, pe_tile, gamma, beta, W, b):
    def _half(e, pe, g, bt, w, bias):
        e = e + pe
        mu = jnp.mean(e, axis=-1, keepdims=True)
        var = jnp.mean(e * e, axis=-1, keepdims=True) - mu * mu
        n = (e - mu) * lax.rsqrt(var + 1e-5)
        n = n * g + bt
        return jnp.dot(n, w, preferred_element_type=jnp.float32) + bias

    def body(e_ref, pe_ref, g_ref, bt_ref, w_ref, b_ref, o_ref):
        pe = pe_ref[...]
        g = g_ref[...]
        bt = bt_ref[...]
        w = w_ref[...]
        bias = b_ref[...]
        o_ref[0] = _half(e_ref[:, :D], pe, g, bt, w, bias)
        o_ref[1] = _half(e_ref[:, D:], pe, g, bt, w, bias)

    return pl.pallas_call(
        body,
        grid=(NSTEPS,),
        in_specs=[
            pl.BlockSpec((TB, P), lambda i: (i, 0)),
            pl.BlockSpec((TB, D), lambda i: (0, 0)),
            pl.BlockSpec((1, D), lambda i: (0, 0)),
            pl.BlockSpec((1, D), lambda i: (0, 0)),
            pl.BlockSpec((D, P), lambda i: (0, 0)),
            pl.BlockSpec((1, P), lambda i: (0, 0)),
        ],
        out_specs=pl.BlockSpec((2, TB, P), lambda i: (0, i, 0)),
        out_shape=jax.ShapeDtypeStruct((2, HALF, P), jnp.float32),
    )(emb2, pe_tile, gamma.reshape(1, D), beta.reshape(1, D), W,
      b.reshape(1, P))


def kernel(x, table, gamma, beta, W, b):
    x_pad = jnp.pad(x.astype(jnp.int32), ((0, 0), (0, SPAD - S)))
    idx_flat = x_pad.reshape(B * SPAD)
    emb2 = _sc_gather(table, idx_flat)
    pe_tile = jnp.asarray(_PE_TILE)
    out = _tc_ln_proj(emb2, pe_tile, gamma, beta, W, b)
    return out.reshape(B, S, P)


# speedup vs baseline: 2.6299x; 2.6299x over previous
import functools
import math

import numpy as np
import jax
import jax.numpy as jnp
from jax import lax
from jax.experimental import pallas as pl
from jax.experimental.pallas import tpu as pltpu
from jax.experimental.pallas import tpu_sc as plsc

VOCAB = 1000000
D = 64
P = 128
B = 1024
S = 200
NTOK = B * S

NC, NS = 2, 16
NW = NC * NS
TOK_PER_W = NTOK // NW
CHUNK = 640
NCHUNK = TOK_PER_W // CHUNK


def _sc_gather(table_wide, idx_flat):
    mesh = plsc.VectorSubcoreMesh(core_axis_name="c", subcore_axis_name="s")

    @functools.partial(
        pl.kernel,
        mesh=mesh,
        out_type=jax.ShapeDtypeStruct((NTOK, P), jnp.float32),
        scratch_types=[
            pltpu.VMEM((CHUNK,), jnp.int32),
            pltpu.VMEM((CHUNK, P), jnp.float32),
            pltpu.SemaphoreType.DMA,
        ],
        compiler_params=pltpu.CompilerParams(use_tc_tiling_on_sc=False),
    )
    def k(table_hbm, idx_hbm, out_hbm, idx_v, rows_v, sem):
        wid = lax.axis_index("s") * NC + lax.axis_index("c")
        base = wid * TOK_PER_W

        def body(i, carry):
            off = base + i * CHUNK
            pltpu.sync_copy(idx_hbm.at[pl.ds(off, CHUNK)], idx_v)
            pltpu.async_copy(table_hbm.at[idx_v], rows_v, sem).wait()
            pltpu.sync_copy(rows_v, out_hbm.at[pl.ds(off, CHUNK)])
            return carry

        lax.fori_loop(0, NCHUNK, body, 0)

    return k(table_wide, idx_flat)


TB = 1600
NSTEPS = NTOK // TB


def _positional_encoding_np(seq_len, embed_dim):
    position = np.arange(0, seq_len, dtype=np.float32)[:, None]
    div_term = np.exp(
        np.arange(0, embed_dim, 2, dtype=np.float32)
        * (-math.log(10000.0) / embed_dim))
    pe = np.zeros((seq_len, embed_dim), dtype=np.float32)
    pe[:, 0::2] = np.sin(position * div_term)
    pe[:, 1::2] = np.cos(position * div_term)
    return pe


_PE_TILE = np.pad(np.tile(_positional_encoding_np(S, D), (TB // S, 1)),
                  ((0, 0), (0, P - D)))


def _tc_ln_proj(emb, pe_tile, wf, bias):
    inv_d = 1.0 / D

    def body(e_ref, pe_ref, w_ref, b_ref, o_ref):
        e = e_ref[...] + pe_ref[...]
        mu = jnp.sum(e, axis=-1, keepdims=True) * inv_d
        var = jnp.sum(e * e, axis=-1, keepdims=True) * inv_d - mu * mu
        n = (e - mu) * lax.rsqrt(var + 1e-5)
        o_ref[...] = (
            jnp.dot(n, w_ref[...], preferred_element_type=jnp.float32)
            + b_ref[...])

    return pl.pallas_call(
        body,
        grid=(NSTEPS,),
        in_specs=[
            pl.BlockSpec((TB, P), lambda i: (i, 0)),
            pl.BlockSpec((TB, P), lambda i: (0, 0)),
            pl.BlockSpec((P, P), lambda i: (0, 0)),
            pl.BlockSpec((1, P), lambda i: (0, 0)),
        ],
        out_specs=pl.BlockSpec((TB, P), lambda i: (i, 0)),
        out_shape=jax.ShapeDtypeStruct((NTOK, P), jnp.float32),
    )(emb, pe_tile, wf, bias)


def kernel(x, table, gamma, beta, W, b):
    table_wide = jnp.pad(table, ((0, 0), (0, P - D)))
    idx_flat = x.reshape(NTOK).astype(jnp.int32)
    emb = _sc_gather(table_wide, idx_flat)
    pe_tile = jnp.asarray(_PE_TILE)
    wf = jnp.pad(gamma[:, None] * W, ((0, P - D), (0, 0)))
    bias = (beta @ W + b).reshape(1, P)
    out = _tc_ln_proj(emb, pe_tile, wf, bias)
    return out.reshape(B, S, P)


# speedup vs baseline: 4.1730x; 1.5867x over previous
import functools
import math

import numpy as np
import jax
import jax.numpy as jnp
from jax import lax
from jax.experimental import pallas as pl
from jax.experimental.pallas import tpu as pltpu
from jax.experimental.pallas import tpu_sc as plsc

VOCAB = 1000000
D = 64
P = 128
B = 1024
S = 200
NTOK = B * S

VB = 16384
NPREP = -(-VOCAB // VB)


def _tc_prep(table_t):
    def body(t_ref, o_ref):
        et = jnp.transpose(t_ref[...], (1, 0))
        o_ref[:, :D] = et
        o_ref[:, D:] = jnp.zeros((VB, P - D), jnp.float32)

    return pl.pallas_call(
        body,
        grid=(NPREP,),
        in_specs=[pl.BlockSpec((D, VB), lambda i: (0, i))],
        out_specs=pl.BlockSpec((VB, P), lambda i: (i, 0)),
        out_shape=jax.ShapeDtypeStruct((VOCAB, P), jnp.float32),
    )(table_t)


NC, NS = 2, 16
NW = NC * NS
TOK_PER_W = NTOK // NW
CHUNK = 640
NCHUNK = TOK_PER_W // CHUNK


def _sc_gather(table_wide, idx_flat):
    mesh = plsc.VectorSubcoreMesh(core_axis_name="c", subcore_axis_name="s")

    @functools.partial(
        pl.kernel,
        mesh=mesh,
        out_type=jax.ShapeDtypeStruct((NTOK, P), jnp.float32),
        scratch_types=[
            pltpu.VMEM((CHUNK,), jnp.int32),
            pltpu.VMEM((CHUNK, P), jnp.float32),
            pltpu.SemaphoreType.DMA,
        ],
        compiler_params=pltpu.CompilerParams(use_tc_tiling_on_sc=False),
    )
    def k(table_hbm, idx_hbm, out_hbm, idx_v, rows_v, sem):
        wid = lax.axis_index("s") * NC + lax.axis_index("c")
        base = wid * TOK_PER_W

        def body(i, carry):
            off = base + i * CHUNK
            pltpu.sync_copy(idx_hbm.at[pl.ds(off, CHUNK)], idx_v)
            pltpu.async_copy(table_hbm.at[idx_v], rows_v, sem).wait()
            pltpu.sync_copy(rows_v, out_hbm.at[pl.ds(off, CHUNK)])
            return carry

        lax.fori_loop(0, NCHUNK, body, 0)

    return k(table_wide, idx_flat)


TB = 1600
NSTEPS = NTOK // TB


def _positional_encoding_np(seq_len, embed_dim):
    position = np.arange(0, seq_len, dtype=np.float32)[:, None]
    div_term = np.exp(
        np.arange(0, embed_dim, 2, dtype=np.float32)
        * (-math.log(10000.0) / embed_dim))
    pe = np.zeros((seq_len, embed_dim), dtype=np.float32)
    pe[:, 0::2] = np.sin(position * div_term)
    pe[:, 1::2] = np.cos(position * div_term)
    return pe


_PE_TILE = np.pad(np.tile(_positional_encoding_np(S, D), (TB // S, 1)),
                  ((0, 0), (0, P - D)))


def _tc_ln_proj(emb, pe_tile, wf, bias):
    inv_d = 1.0 / D

    def body(e_ref, pe_ref, w_ref, b_ref, o_ref):
        e = e_ref[...] + pe_ref[...]
        mu = jnp.sum(e, axis=-1, keepdims=True) * inv_d
        var = jnp.sum(e * e, axis=-1, keepdims=True) * inv_d - mu * mu
        n = (e - mu) * lax.rsqrt(var + 1e-5)
        o_ref[...] = (
            jnp.dot(n, w_ref[...], preferred_element_type=jnp.float32)
            + b_ref[...])

    return pl.pallas_call(
        body,
        grid=(NSTEPS,),
        in_specs=[
            pl.BlockSpec((TB, P), lambda i: (i, 0)),
            pl.BlockSpec((TB, P), lambda i: (0, 0)),
            pl.BlockSpec((P, P), lambda i: (0, 0)),
            pl.BlockSpec((1, P), lambda i: (0, 0)),
        ],
        out_specs=pl.BlockSpec((TB, P), lambda i: (i, 0)),
        out_shape=jax.ShapeDtypeStruct((NTOK, P), jnp.float32),
    )(emb, pe_tile, wf, bias)


def kernel(x, table, gamma, beta, W, b):
    table_wide = _tc_prep(jnp.transpose(table))
    idx_flat = x.reshape(NTOK).astype(jnp.int32)
    emb = _sc_gather(table_wide, idx_flat)
    pe_tile = jnp.asarray(_PE_TILE)
    wf = jnp.pad(gamma[:, None] * W, ((0, P - D), (0, 0)))
    bias = (beta @ W + b).reshape(1, P)
    out = _tc_ln_proj(emb, pe_tile, wf, bias)
    return out.reshape(B, S, P)


# speedup vs baseline: 4.4991x; 1.0782x over previous
import functools
import math

import numpy as np
import jax
import jax.numpy as jnp
from jax import lax
from jax.experimental import pallas as pl
from jax.experimental.pallas import tpu as pltpu
from jax.experimental.pallas import tpu_sc as plsc

VOCAB = 1000000
D = 64
P = 128
B = 1024
S = 200
NTOK = B * S

VB = 16384
NPREP = -(-VOCAB // VB)


def _tc_prep(table_t):
    def body(t_ref, o_ref):
        et = jnp.transpose(t_ref[...], (1, 0))
        o_ref[:, :D] = et
        o_ref[:, D:] = jnp.zeros((VB, P - D), jnp.float32)

    return pl.pallas_call(
        body,
        grid=(NPREP,),
        in_specs=[pl.BlockSpec((D, VB), lambda i: (0, i))],
        out_specs=pl.BlockSpec((VB, P), lambda i: (i, 0)),
        out_shape=jax.ShapeDtypeStruct((VOCAB, P), jnp.float32),
    )(table_t)


NC, NS = 2, 16
NW = NC * NS
TOK_PER_W = NTOK // NW
CHUNK = 640
NCHUNK = TOK_PER_W // CHUNK


def _sc_gather(table_wide, idx_flat):
    mesh = plsc.VectorSubcoreMesh(core_axis_name="c", subcore_axis_name="s")

    @functools.partial(
        pl.kernel,
        mesh=mesh,
        out_type=jax.ShapeDtypeStruct((NTOK, P), jnp.float32),
        scratch_types=[
            pltpu.VMEM((CHUNK,), jnp.int32),
            pltpu.VMEM((CHUNK, P), jnp.float32),
            pltpu.SemaphoreType.DMA,
        ],
        compiler_params=pltpu.CompilerParams(use_tc_tiling_on_sc=False),
    )
    def k(table_hbm, idx_hbm, out_hbm, idx_v, rows_v, sem):
        wid = lax.axis_index("s") * NC + lax.axis_index("c")
        base = wid * TOK_PER_W

        def body(i, carry):
            off = base + i * CHUNK
            pltpu.sync_copy(idx_hbm.at[pl.ds(off, CHUNK)], idx_v)
            pltpu.async_copy(table_hbm.at[idx_v], rows_v, sem).wait()
            pltpu.sync_copy(rows_v, out_hbm.at[pl.ds(off, CHUNK)])
            return carry

        lax.fori_loop(0, NCHUNK, body, 0)

    return k(table_wide, idx_flat)


TB = 3200
NSTEPS = NTOK // TB


def _positional_encoding_np(seq_len, embed_dim):
    position = np.arange(0, seq_len, dtype=np.float32)[:, None]
    div_term = np.exp(
        np.arange(0, embed_dim, 2, dtype=np.float32)
        * (-math.log(10000.0) / embed_dim))
    pe = np.zeros((seq_len, embed_dim), dtype=np.float32)
    pe[:, 0::2] = np.sin(position * div_term)
    pe[:, 1::2] = np.cos(position * div_term)
    return pe


_PE_TILE = np.pad(np.tile(_positional_encoding_np(S, D), (TB // S, 1)),
                  ((0, 0), (0, P - D)))


def _tc_ln_proj(emb, pe_tile, wf, bias):
    inv_d = 1.0 / D

    def body(e_ref, pe_ref, w_ref, b_ref, o_ref):
        e = e_ref[...] + pe_ref[...]
        mu = jnp.sum(e, axis=-1, keepdims=True) * inv_d
        var = jnp.sum(e * e, axis=-1, keepdims=True) * inv_d - mu * mu
        n = (e - mu) * lax.rsqrt(var + 1e-5)
        o_ref[...] = (
            jnp.dot(n, w_ref[...], preferred_element_type=jnp.float32)
            + b_ref[...])

    return pl.pallas_call(
        body,
        grid=(NSTEPS,),
        in_specs=[
            pl.BlockSpec((TB, P), lambda i: (i, 0)),
            pl.BlockSpec((TB, P), lambda i: (0, 0)),
            pl.BlockSpec((P, P), lambda i: (0, 0)),
            pl.BlockSpec((1, P), lambda i: (0, 0)),
        ],
        out_specs=pl.BlockSpec((TB, P), lambda i: (i, 0)),
        out_shape=jax.ShapeDtypeStruct((NTOK, P), jnp.float32),
    )(emb, pe_tile, wf, bias)


def kernel(x, table, gamma, beta, W, b):
    table_wide = _tc_prep(jnp.transpose(table))
    idx_flat = x.reshape(NTOK).astype(jnp.int32)
    emb = _sc_gather(table_wide, idx_flat)
    pe_tile = jnp.asarray(_PE_TILE)
    wf = jnp.pad(gamma[:, None] * W, ((0, P - D), (0, 0)))
    bias = (beta @ W + b).reshape(1, P)
    out = _tc_ln_proj(emb, pe_tile, wf, bias)
    return out.reshape(B, S, P)
